# baseline (speedup 1.0000x reference)
import functools

import jax
import jax.numpy as jnp
from jax import lax
from jax.experimental import pallas as pl
from jax.experimental.pallas import tpu as pltpu
from jax.experimental.pallas import tpu_sc as plsc

B = 4096
S = 200
D = 64
H = 256
C = 50

NC, NS, L = 2, 16, 16
NW = NC * NS
SPW = B // NW
CH0 = 128
CH1 = S - CH0

_mesh = plsc.VectorSubcoreMesh(core_axis_name="c", subcore_axis_name="s")


@functools.partial(
    pl.kernel,
    mesh=_mesh,
    compiler_params=pltpu.CompilerParams(use_tc_tiling_on_sc=False),
    out_type=jax.ShapeDtypeStruct((B, D), jnp.float32),
    scratch_types=[
        pltpu.VMEM((SPW * S,), jnp.int32),
        pltpu.VMEM((S, D), jnp.float32),
        pltpu.VMEM((SPW, D), jnp.float32),
        pltpu.SemaphoreType.DMA,
    ],
)
def _pool(x_hbm, emb_hbm, out_hbm, idx_v, rows_v, res_v, sem):
    wid = lax.axis_index("s") * NC + lax.axis_index("c")
    base = wid * SPW
    pltpu.sync_copy(x_hbm.at[pl.ds(base * S, SPW * S)], idx_v)

    def sample_body(s, carry):
        off = s * S
        cp0 = pltpu.async_copy(
            emb_hbm.at[idx_v.at[pl.ds(off, CH0)]], rows_v.at[pl.ds(0, CH0)], sem)
        cp1 = pltpu.async_copy(
            emb_hbm.at[idx_v.at[pl.ds(off + CH0, CH1)]],
            rows_v.at[pl.ds(CH0, CH1)], sem)
        cp0.wait()
        cp1.wait()

        def row_body(r, accs):
            return tuple(
                accs[c] + rows_v[r, pl.ds(c * L, L)] for c in range(D // L))

        accs = lax.fori_loop(
            0, S, row_body,
            tuple(jnp.zeros((L,), jnp.float32) for _ in range(D // L)))
        for c in range(D // L):
            res_v[s, pl.ds(c * L, L)] = accs[c] * (1.0 / S)
        return carry

    lax.fori_loop(0, SPW, sample_body, 0)
    pltpu.sync_copy(res_v, out_hbm.at[pl.ds(base, SPW)])


def _mlp_body(p_ref, w1_ref, b1_ref, w2_ref, b2_ref, o_ref):
    h = jnp.dot(p_ref[...], w1_ref[...], preferred_element_type=jnp.float32)
    h = jnp.maximum(h + b1_ref[...], 0.0)
    o_ref[...] = (
        jnp.dot(h, w2_ref[...], preferred_element_type=jnp.float32)
        + b2_ref[...])


_BB = 1024


@jax.jit
def _mlp(pooled, W1, b1, W2p, b2p):
    return pl.pallas_call(
        _mlp_body,
        grid=(B // _BB,),
        in_specs=[
            pl.BlockSpec((_BB, D), lambda i: (i, 0)),
            pl.BlockSpec((D, H), lambda i: (0, 0)),
            pl.BlockSpec((1, H), lambda i: (0, 0)),
            pl.BlockSpec((H, D), lambda i: (0, 0)),
            pl.BlockSpec((1, D), lambda i: (0, 0)),
        ],
        out_specs=pl.BlockSpec((_BB, D), lambda i: (i, 0)),
        out_shape=jax.ShapeDtypeStruct((B, D), jnp.float32),
    )(pooled, W1, b1, W2p, b2p)


def kernel(x, emb, W1, b1, W2, b2):
    x_flat = x.reshape(-1).astype(jnp.int32)
    pooled = _pool(x_flat, emb)
    W2p = jnp.pad(W2, ((0, 0), (0, D - C)))
    b2p = jnp.pad(b2, (0, D - C)).reshape(1, D)
    out = _mlp(pooled, W1, b1.reshape(1, H), W2p, b2p)
    return out[:, :C]


# speedup vs baseline: 1.2375x; 1.2375x over previous
import functools

import jax
import jax.numpy as jnp
from jax import lax
from jax.experimental import pallas as pl
from jax.experimental.pallas import tpu as pltpu
from jax.experimental.pallas import tpu_sc as plsc

B = 4096
S = 200
D = 64
H = 256
C = 50

NC, NS, L = 2, 16, 16
NW = NC * NS
SPW = B // NW
CH0 = 128
CH1 = S - CH0

_mesh = plsc.VectorSubcoreMesh(core_axis_name="c", subcore_axis_name="s")


NBUF = 4
RU = 8


@functools.partial(
    pl.kernel,
    mesh=_mesh,
    compiler_params=pltpu.CompilerParams(use_tc_tiling_on_sc=False),
    out_type=jax.ShapeDtypeStruct((B, D), jnp.float32),
    scratch_types=[
        pltpu.VMEM((SPW * S,), jnp.int32),
        pltpu.VMEM((NBUF, S, D), jnp.float32),
        pltpu.VMEM((SPW, D), jnp.float32),
        [pltpu.SemaphoreType.DMA] * NBUF,
    ],
)
def _pool(x_hbm, emb_hbm, out_hbm, idx_v, rows_v, res_v, sems):
    wid = lax.axis_index("s") * NC + lax.axis_index("c")
    base = wid * SPW
    pltpu.sync_copy(x_hbm.at[pl.ds(base * S, SPW * S)], idx_v)

    def issue(s, b):
        off = s * S
        pltpu.async_copy(
            emb_hbm.at[idx_v.at[pl.ds(off, CH0)]],
            rows_v.at[b].at[pl.ds(0, CH0)], sems[b])
        pltpu.async_copy(
            emb_hbm.at[idx_v.at[pl.ds(off + CH0, CH1)]],
            rows_v.at[b].at[pl.ds(CH0, CH1)], sems[b])

    def drain(s, b):
        off = s * S
        pltpu.make_async_copy(
            emb_hbm.at[idx_v.at[pl.ds(off, CH0)]],
            rows_v.at[b].at[pl.ds(0, CH0)], sems[b]).wait()
        pltpu.make_async_copy(
            emb_hbm.at[idx_v.at[pl.ds(off + CH0, CH1)]],
            rows_v.at[b].at[pl.ds(CH0, CH1)], sems[b]).wait()

    for b in range(NBUF):
        issue(b, b)

    def round_body(g, carry):
        for b in range(NBUF):
            s = g * NBUF + b
            drain(s, b)

            def row_body(i, accs):
                r0 = i * RU
                out = list(accs)
                for j in range(RU):
                    for c in range(D // L):
                        out[c] = out[c] + rows_v[b, r0 + j, pl.ds(c * L, L)]
                return tuple(out)

            accs = lax.fori_loop(
                0, S // RU, row_body,
                tuple(jnp.zeros((L,), jnp.float32) for _ in range(D // L)))
            for c in range(D // L):
                res_v[s, pl.ds(c * L, L)] = accs[c] * (1.0 / S)

            @pl.when(s + NBUF < SPW)
            def _():
                issue(s + NBUF, b)
        return carry

    lax.fori_loop(0, SPW // NBUF, round_body, 0)
    pltpu.sync_copy(res_v, out_hbm.at[pl.ds(base, SPW)])


def _mlp_body(p_ref, w1_ref, b1_ref, w2_ref, b2_ref, o_ref):
    h = jnp.dot(p_ref[...], w1_ref[...], preferred_element_type=jnp.float32)
    h = jnp.maximum(h + b1_ref[...], 0.0)
    o_ref[...] = (
        jnp.dot(h, w2_ref[...], preferred_element_type=jnp.float32)
        + b2_ref[...])


_BB = 1024


@jax.jit
def _mlp(pooled, W1, b1, W2p, b2p):
    return pl.pallas_call(
        _mlp_body,
        grid=(B // _BB,),
        in_specs=[
            pl.BlockSpec((_BB, D), lambda i: (i, 0)),
            pl.BlockSpec((D, H), lambda i: (0, 0)),
            pl.BlockSpec((1, H), lambda i: (0, 0)),
            pl.BlockSpec((H, D), lambda i: (0, 0)),
            pl.BlockSpec((1, D), lambda i: (0, 0)),
        ],
        out_specs=pl.BlockSpec((_BB, D), lambda i: (i, 0)),
        out_shape=jax.ShapeDtypeStruct((B, D), jnp.float32),
    )(pooled, W1, b1, W2p, b2p)


def kernel(x, emb, W1, b1, W2, b2):
    x_flat = x.reshape(-1).astype(jnp.int32)
    pooled = _pool(x_flat, emb)
    W2p = jnp.pad(W2, ((0, 0), (0, D - C)))
    b2p = jnp.pad(b2, (0, D - C)).reshape(1, D)
    out = _mlp(pooled, W1, b1.reshape(1, H), W2p, b2p)
    return out[:, :C]


# speedup vs baseline: 1.2404x; 1.0024x over previous
import functools

import jax
import jax.numpy as jnp
from jax import lax
from jax.experimental import pallas as pl
from jax.experimental.pallas import tpu as pltpu
from jax.experimental.pallas import tpu_sc as plsc

B = 4096
S = 200
D = 64
H = 256
C = 50

NC, NS, L = 2, 16, 16
NW = NC * NS
SPW = B // NW
CH0 = 128
CH1 = S - CH0

_mesh = plsc.VectorSubcoreMesh(core_axis_name="c", subcore_axis_name="s")


NBUF = 4
RU = 8


@functools.partial(
    pl.kernel,
    mesh=_mesh,
    compiler_params=pltpu.CompilerParams(use_tc_tiling_on_sc=False),
    out_type=jax.ShapeDtypeStruct((B, D), jnp.float32),
    scratch_types=[
        pltpu.VMEM((SPW, S), jnp.int32),
        pltpu.VMEM((NBUF, S, D), jnp.float32),
        pltpu.VMEM((SPW, D), jnp.float32),
        [pltpu.SemaphoreType.DMA] * NBUF,
    ],
)
def _pool(x_hbm, emb_hbm, out_hbm, idx_v, rows_v, res_v, sems):
    wid = lax.axis_index("s") * NC + lax.axis_index("c")
    base = wid * SPW
    pltpu.sync_copy(x_hbm.at[pl.ds(base, SPW)], idx_v)

    def issue(s, b):
        pltpu.async_copy(
            emb_hbm.at[idx_v.at[s, pl.ds(0, CH0)]],
            rows_v.at[b].at[pl.ds(0, CH0)], sems[b])
        pltpu.async_copy(
            emb_hbm.at[idx_v.at[s, pl.ds(CH0, CH1)]],
            rows_v.at[b].at[pl.ds(CH0, CH1)], sems[b])

    def drain(s, b):
        pltpu.make_async_copy(
            emb_hbm.at[idx_v.at[s, pl.ds(0, CH0)]],
            rows_v.at[b].at[pl.ds(0, CH0)], sems[b]).wait()
        pltpu.make_async_copy(
            emb_hbm.at[idx_v.at[s, pl.ds(CH0, CH1)]],
            rows_v.at[b].at[pl.ds(CH0, CH1)], sems[b]).wait()

    for b in range(NBUF):
        issue(b, b)

    def round_body(g, carry):
        for b in range(NBUF):
            s = g * NBUF + b
            drain(s, b)

            def row_body(i, accs):
                r0 = i * RU
                out = list(accs)
                for j in range(RU):
                    for c in range(D // L):
                        out[c] = out[c] + rows_v[b, r0 + j, pl.ds(c * L, L)]
                return tuple(out)

            accs = lax.fori_loop(
                0, S // RU, row_body,
                tuple(jnp.zeros((L,), jnp.float32) for _ in range(D // L)))
            for c in range(D // L):
                res_v[s, pl.ds(c * L, L)] = accs[c] * (1.0 / S)

            @pl.when(s + NBUF < SPW)
            def _():
                issue(s + NBUF, b)
        return carry

    lax.fori_loop(0, SPW // NBUF, round_body, 0)
    pltpu.sync_copy(res_v, out_hbm.at[pl.ds(base, SPW)])


def _mlp_body(p_ref, w1_ref, b1_ref, w2_ref, b2_ref, o_ref):
    h = jnp.dot(p_ref[...], w1_ref[...], preferred_element_type=jnp.float32)
    h = jnp.maximum(h + b1_ref[...], 0.0)
    o_ref[...] = (
        jnp.dot(h, w2_ref[...], preferred_element_type=jnp.float32)
        + b2_ref[...])


_BB = 1024


@jax.jit
def _mlp(pooled, W1, b1, W2p, b2p):
    return pl.pallas_call(
        _mlp_body,
        grid=(B // _BB,),
        in_specs=[
            pl.BlockSpec((_BB, D), lambda i: (i, 0)),
            pl.BlockSpec((D, H), lambda i: (0, 0)),
            pl.BlockSpec((1, H), lambda i: (0, 0)),
            pl.BlockSpec((H, D), lambda i: (0, 0)),
            pl.BlockSpec((1, D), lambda i: (0, 0)),
        ],
        out_specs=pl.BlockSpec((_BB, D), lambda i: (i, 0)),
        out_shape=jax.ShapeDtypeStruct((B, D), jnp.float32),
    )(pooled, W1, b1, W2p, b2p)


def kernel(x, emb, W1, b1, W2, b2):
    pooled = _pool(x.astype(jnp.int32), emb)
    W2p = jnp.pad(W2, ((0, 0), (0, D - C)))
    b2p = jnp.pad(b2, (0, D - C)).reshape(1, D)
    out = _mlp(pooled, W1, b1.reshape(1, H), W2p, b2p)
    return out[:, :C]
